# Initial kernel scaffold; baseline (speedup 1.0000x reference)
import jax
import jax.numpy as jnp
from jax.experimental import pallas as pl


def kernel(wavs, W_enc, W_in_0, codebook_0, W_out_0, W_in_1, codebook_1, W_out_1, W_in_2, codebook_2, W_out_2):
    raise NotImplementedError("write your pallas kernel here")


# speedup vs baseline: 1.4729x; 1.4729x over previous
import jax
import jax.numpy as jnp
from jax.experimental import pallas as pl

_B, _T = 16, 160000
_HOP = 320
_D = 1024
_CB_SIZE = 1024
_CB_DIM = 8
_F = _T // _HOP
_N = _B * _F
_R = 1000

_HI = jax.lax.Precision.DEFAULT


def _dot(a, b):
    return jax.lax.dot_general(a, b, (((1,), (0,)), ((), ())),
                               precision=_HI, preferred_element_type=jnp.float32)


def _dot_t(a, b):
    return jax.lax.dot_general(a, b, (((1,), (1,)), ((), ())),
                               precision=_HI, preferred_element_type=jnp.float32)


def _rvq_kernel(wav_ref, W_enc_ref,
                Wi0_ref, cb0_ref, Wo0_ref,
                Wi1_ref, cb1_ref, Wo1_ref,
                Wi2_ref, cb2_ref, Wo2_ref,
                out_ref):
    z = _dot(wav_ref[...], W_enc_ref[...])
    residual = z
    zq = jnp.zeros_like(z)
    iota = jax.lax.broadcasted_iota(jnp.int32, (_R, _CB_SIZE), 1)
    for Wi_ref, cb_ref, Wo_ref in ((Wi0_ref, cb0_ref, Wo0_ref),
                                   (Wi1_ref, cb1_ref, Wo1_ref),
                                   (Wi2_ref, cb2_ref, Wo2_ref)):
        cb = cb_ref[...]
        z_e = _dot(residual, Wi_ref[...])
        enc = z_e / (jnp.sqrt(jnp.sum(z_e * z_e, -1, keepdims=True)) + 1e-8)
        cbn = cb / (jnp.sqrt(jnp.sum(cb * cb, -1, keepdims=True)) + 1e-8)
        dist = (jnp.sum(enc * enc, -1, keepdims=True)
                - 2.0 * _dot_t(enc, cbn)
                + jnp.sum(cbn * cbn, -1)[None, :])
        idx = jnp.argmin(dist, axis=-1)
        onehot = (iota == idx[:, None]).astype(jnp.float32)
        q = _dot(onehot, cb)
        out_i = _dot(q, Wo_ref[...])
        zq = zq + out_i
        residual = residual - out_i
    out_ref[...] = zq


def kernel(wavs, W_enc, W_in_0, codebook_0, W_out_0,
           W_in_1, codebook_1, W_out_1, W_in_2, codebook_2, W_out_2):
    rows = wavs.reshape(_N, _HOP)
    full = lambda shape: pl.BlockSpec(shape, lambda i: (0, 0))
    out = pl.pallas_call(
        _rvq_kernel,
        grid=(_N // _R,),
        in_specs=[
            pl.BlockSpec((_R, _HOP), lambda i: (i, 0)),
            full((_HOP, _D)),
            full((_D, _CB_DIM)), full((_CB_SIZE, _CB_DIM)), full((_CB_DIM, _D)),
            full((_D, _CB_DIM)), full((_CB_SIZE, _CB_DIM)), full((_CB_DIM, _D)),
            full((_D, _CB_DIM)), full((_CB_SIZE, _CB_DIM)), full((_CB_DIM, _D)),
        ],
        out_specs=pl.BlockSpec((_R, _D), lambda i: (i, 0)),
        out_shape=jax.ShapeDtypeStruct((_N, _D), jnp.float32),
    )(rows, W_enc,
      W_in_0, codebook_0, W_out_0,
      W_in_1, codebook_1, W_out_1,
      W_in_2, codebook_2, W_out_2)
    return out.reshape(_B, _F, _D)


# speedup vs baseline: 1.4751x; 1.0015x over previous
import jax
import jax.numpy as jnp
from jax.experimental import pallas as pl

_B, _T = 16, 160000
_HOP = 320
_D = 1024
_CB_SIZE = 1024
_CB_DIM = 8
_F = _T // _HOP
_N = _B * _F
_R = 1000

_HI = jax.lax.Precision.DEFAULT


def _dot(a, b):
    return jax.lax.dot_general(a, b, (((1,), (0,)), ((), ())),
                               precision=_HI, preferred_element_type=jnp.float32)


def _dot_t(a, b):
    return jax.lax.dot_general(a, b, (((1,), (1,)), ((), ())),
                               precision=_HI, preferred_element_type=jnp.float32)


def _rvq_kernel(wav_ref, W_enc_ref,
                Wi0_ref, cb0_ref, Wo0_ref,
                Wi1_ref, cb1_ref, Wo1_ref,
                Wi2_ref, cb2_ref, Wo2_ref,
                out_ref):
    z = _dot(wav_ref[...], W_enc_ref[...])
    residual = z
    iota = jax.lax.broadcasted_iota(jnp.int32, (_R, _CB_SIZE), 1)
    for stage, (Wi_ref, cb_ref, Wo_ref) in enumerate((
            (Wi0_ref, cb0_ref, Wo0_ref),
            (Wi1_ref, cb1_ref, Wo1_ref),
            (Wi2_ref, cb2_ref, Wo2_ref))):
        cb = cb_ref[...]
        z_e = _dot(residual, Wi_ref[...])
        enc = z_e / (jnp.sqrt(jnp.sum(z_e * z_e, -1, keepdims=True)) + 1e-8)
        cbn = cb / (jnp.sqrt(jnp.sum(cb * cb, -1, keepdims=True)) + 1e-8)
        dist = (jnp.sum(enc * enc, -1, keepdims=True)
                - 2.0 * _dot_t(enc, cbn)
                + jnp.sum(cbn * cbn, -1)[None, :])
        idx = jnp.argmin(dist, axis=-1)
        onehot = (iota == idx[:, None]).astype(jnp.float32)
        q = _dot(onehot, cb)
        out_i = _dot(q, Wo_ref[...])
        if stage < 2:
            residual = residual - out_i
        else:
            out_ref[...] = (z - residual) + out_i


def kernel(wavs, W_enc, W_in_0, codebook_0, W_out_0,
           W_in_1, codebook_1, W_out_1, W_in_2, codebook_2, W_out_2):
    rows = wavs.reshape(_N, _HOP)
    full = lambda shape: pl.BlockSpec(shape, lambda i: (0, 0))
    out = pl.pallas_call(
        _rvq_kernel,
        grid=(_N // _R,),
        in_specs=[
            pl.BlockSpec((_R, _HOP), lambda i: (i, 0)),
            full((_HOP, _D)),
            full((_D, _CB_DIM)), full((_CB_SIZE, _CB_DIM)), full((_CB_DIM, _D)),
            full((_D, _CB_DIM)), full((_CB_SIZE, _CB_DIM)), full((_CB_DIM, _D)),
            full((_D, _CB_DIM)), full((_CB_SIZE, _CB_DIM)), full((_CB_DIM, _D)),
        ],
        out_specs=pl.BlockSpec((_R, _D), lambda i: (i, 0)),
        out_shape=jax.ShapeDtypeStruct((_N, _D), jnp.float32),
    )(rows, W_enc,
      W_in_0, codebook_0, W_out_0,
      W_in_1, codebook_1, W_out_1,
      W_in_2, codebook_2, W_out_2)
    return out.reshape(_B, _F, _D)


# speedup vs baseline: 1.6516x; 1.1196x over previous
import jax
import jax.numpy as jnp
from jax.experimental import pallas as pl

_B, _T = 16, 160000
_HOP = 320
_D = 1024
_CB_SIZE = 1024
_CB_DIM = 8
_F = _T // _HOP
_N = _B * _F
_R = _F

_HI = jax.lax.Precision.DEFAULT


def _dot(a, b):
    return jax.lax.dot_general(a, b, (((1,), (0,)), ((), ())),
                               precision=_HI, preferred_element_type=jnp.float32)


def _dot_t(a, b):
    return jax.lax.dot_general(a, b, (((1,), (1,)), ((), ())),
                               precision=_HI, preferred_element_type=jnp.float32)


def _rvq_kernel(wav_ref, W_enc_ref,
                Wi0_ref, cb0_ref, Wo0_ref,
                Wi1_ref, cb1_ref, Wo1_ref,
                Wi2_ref, cb2_ref, Wo2_ref,
                out_ref):
    z = _dot(wav_ref[0], W_enc_ref[...])
    residual = z
    iota = jax.lax.broadcasted_iota(jnp.int32, (_R, _CB_SIZE), 1)
    for stage, (Wi_ref, cb_ref, Wo_ref) in enumerate((
            (Wi0_ref, cb0_ref, Wo0_ref),
            (Wi1_ref, cb1_ref, Wo1_ref),
            (Wi2_ref, cb2_ref, Wo2_ref))):
        cb = cb_ref[...]
        z_e = _dot(residual, Wi_ref[...])
        enc = z_e / (jnp.sqrt(jnp.sum(z_e * z_e, -1, keepdims=True)) + 1e-8)
        cbn = cb / (jnp.sqrt(jnp.sum(cb * cb, -1, keepdims=True)) + 1e-8)
        dist = (jnp.sum(enc * enc, -1, keepdims=True)
                - 2.0 * _dot_t(enc, cbn)
                + jnp.sum(cbn * cbn, -1)[None, :])
        idx = jnp.argmin(dist, axis=-1)
        onehot = (iota == idx[:, None]).astype(jnp.float32)
        q = _dot(onehot, cb)
        out_i = _dot(q, Wo_ref[...])
        if stage < 2:
            residual = residual - out_i
        else:
            out_ref[...] = ((z - residual) + out_i)[None]


def kernel(wavs, W_enc, W_in_0, codebook_0, W_out_0,
           W_in_1, codebook_1, W_out_1, W_in_2, codebook_2, W_out_2):
    rows = wavs.reshape(_B, _F, _HOP)
    full = lambda shape: pl.BlockSpec(shape, lambda i: (0, 0))
    out = pl.pallas_call(
        _rvq_kernel,
        grid=(_B,),
        in_specs=[
            pl.BlockSpec((1, _F, _HOP), lambda i: (i, 0, 0)),
            full((_HOP, _D)),
            full((_D, _CB_DIM)), full((_CB_SIZE, _CB_DIM)), full((_CB_DIM, _D)),
            full((_D, _CB_DIM)), full((_CB_SIZE, _CB_DIM)), full((_CB_DIM, _D)),
            full((_D, _CB_DIM)), full((_CB_SIZE, _CB_DIM)), full((_CB_DIM, _D)),
        ],
        out_specs=pl.BlockSpec((1, _F, _D), lambda i: (i, 0, 0)),
        out_shape=jax.ShapeDtypeStruct((_B, _F, _D), jnp.float32),
    )(rows, W_enc,
      W_in_0, codebook_0, W_out_0,
      W_in_1, codebook_1, W_out_1,
      W_in_2, codebook_2, W_out_2)
    return out
